# baseline (speedup 1.0000x reference)
import jax
import jax.numpy as jnp
from jax.experimental import pallas as pl

_ANCHORS = [(116.0, 90.0), (156.0, 198.0), (373.0, 326.0)]
_NUM_ANCHORS = 3
_NUM_CLASSES = 80
_BBOX_ATTRS = 5 + _NUM_CLASSES
_INPUT_SHAPE = (608, 608)


def _decode_kernel(in_ref, out_ref, *, in_h, in_w, stride_w, stride_h):
    hw = in_h * in_w
    a = pl.program_id(1)
    aw8 = jnp.where(a == 0, _ANCHORS[0][0] / stride_w,
                    jnp.where(a == 1, _ANCHORS[1][0] / stride_w,
                              _ANCHORS[2][0] / stride_w))
    ah8 = jnp.where(a == 0, _ANCHORS[0][1] / stride_h,
                    jnp.where(a == 1, _ANCHORS[1][1] / stride_h,
                              _ANCHORS[2][1] / stride_h))

    p = in_ref[0, 0]
    head = p[0:8]
    sig_h = jax.nn.sigmoid(head)
    ex_h = jnp.exp(head)

    r = jax.lax.broadcasted_iota(jnp.int32, (8, hw), 0)
    k = jax.lax.broadcasted_iota(jnp.int32, (8, hw), 1)
    gx = (k % in_w).astype(jnp.float32)
    gy = (k // in_w).astype(jnp.float32)

    val_h = jnp.where(
        r == 0, (sig_h + gx) * stride_w,
        jnp.where(
            r == 1, (sig_h + gy) * stride_h,
            jnp.where(
                r == 2, ex_h * (aw8 * stride_w),
                jnp.where(r == 3, ex_h * (ah8 * stride_h), sig_h))))
    val = jnp.concatenate([val_h, jax.nn.sigmoid(p[8:])], axis=0)
    out_ref[0, 0] = val.T


def kernel(input):
    bs, ch, in_h, in_w = input.shape
    hw = in_h * in_w
    stride_h = _INPUT_SHAPE[0] / in_h
    stride_w = _INPUT_SHAPE[1] / in_w

    x = input.reshape(bs, _NUM_ANCHORS, _BBOX_ATTRS, hw)

    out = pl.pallas_call(
        lambda i_ref, o_ref: _decode_kernel(
            i_ref, o_ref, in_h=in_h, in_w=in_w,
            stride_w=stride_w, stride_h=stride_h),
        grid=(bs, _NUM_ANCHORS),
        in_specs=[pl.BlockSpec((1, 1, _BBOX_ATTRS, hw), lambda b, a: (b, a, 0, 0))],
        out_specs=pl.BlockSpec((1, 1, hw, _BBOX_ATTRS), lambda b, a: (b, a, 0, 0)),
        out_shape=jax.ShapeDtypeStruct((bs, _NUM_ANCHORS, hw, _BBOX_ATTRS), jnp.float32),
    )(x)

    return out.reshape(bs, _NUM_ANCHORS * hw, _BBOX_ATTRS)


# speedup vs baseline: 1.6930x; 1.6930x over previous
import jax
import jax.numpy as jnp
from jax.experimental import pallas as pl

_ANCHORS = [(116.0, 90.0), (156.0, 198.0), (373.0, 326.0)]
_NUM_ANCHORS = 3
_NUM_CLASSES = 80
_BBOX_ATTRS = 5 + _NUM_CLASSES
_INPUT_SHAPE = (608, 608)


def _decode_kernel(in_ref, out_ref, *, in_h, in_w, stride_w, stride_h):
    hw = in_h * in_w
    a = pl.program_id(1)
    aw8 = jnp.where(a == 0, _ANCHORS[0][0] / stride_w,
                    jnp.where(a == 1, _ANCHORS[1][0] / stride_w,
                              _ANCHORS[2][0] / stride_w))
    ah8 = jnp.where(a == 0, _ANCHORS[0][1] / stride_h,
                    jnp.where(a == 1, _ANCHORS[1][1] / stride_h,
                              _ANCHORS[2][1] / stride_h))

    p = in_ref[0].reshape(_BBOX_ATTRS, hw)
    head = p[0:8]
    sig_h = jax.nn.sigmoid(head)
    ex_h = jnp.exp(head)

    r = jax.lax.broadcasted_iota(jnp.int32, (8, hw), 0)
    k = jax.lax.broadcasted_iota(jnp.int32, (8, hw), 1)
    gx = (k % in_w).astype(jnp.float32)
    gy = (k // in_w).astype(jnp.float32)

    val_h = jnp.where(
        r == 0, (sig_h + gx) * stride_w,
        jnp.where(
            r == 1, (sig_h + gy) * stride_h,
            jnp.where(
                r == 2, ex_h * (aw8 * stride_w),
                jnp.where(r == 3, ex_h * (ah8 * stride_h), sig_h))))
    val = jnp.concatenate([val_h, jax.nn.sigmoid(p[8:])], axis=0)
    out_ref[0] = val.T


def kernel(input):
    bs, ch, in_h, in_w = input.shape
    hw = in_h * in_w
    stride_h = _INPUT_SHAPE[0] / in_h
    stride_w = _INPUT_SHAPE[1] / in_w

    out = pl.pallas_call(
        lambda i_ref, o_ref: _decode_kernel(
            i_ref, o_ref, in_h=in_h, in_w=in_w,
            stride_w=stride_w, stride_h=stride_h),
        grid=(bs, _NUM_ANCHORS),
        in_specs=[pl.BlockSpec((1, _BBOX_ATTRS, in_h, in_w),
                               lambda b, a: (b, a, 0, 0))],
        out_specs=pl.BlockSpec((1, hw, _BBOX_ATTRS), lambda b, a: (b, a, 0)),
        out_shape=jax.ShapeDtypeStruct(
            (bs, _NUM_ANCHORS * hw, _BBOX_ATTRS), jnp.float32),
    )(input)
    return out


# speedup vs baseline: 3.0714x; 1.8142x over previous
import jax
import jax.numpy as jnp
from jax.experimental import pallas as pl

_ANCHORS = [(116.0, 90.0), (156.0, 198.0), (373.0, 326.0)]
_NUM_ANCHORS = 3
_NUM_CLASSES = 80
_BBOX_ATTRS = 5 + _NUM_CLASSES
_INPUT_SHAPE = (608, 608)
_C_CHUNK = 17
_B_CHUNK = 8


def _decode_kernel(in0, in1, in2, out_ref, *, in_h, in_w, stride_w, stride_h):
    hw = in_h * in_w
    cc = pl.program_id(1)

    r = jax.lax.broadcasted_iota(jnp.int32, (4, _B_CHUNK, hw), 0)
    k = jax.lax.broadcasted_iota(jnp.int32, (4, _B_CHUNK, hw), 2)
    gx = (k % in_w).astype(jnp.float32)
    gy = (k // in_w).astype(jnp.float32)

    for a, ref in enumerate((in0, in1, in2)):
        aw8 = _ANCHORS[a][0] / stride_w
        ah8 = _ANCHORS[a][1] / stride_h
        t = jnp.transpose(ref[...], (1, 0, 2, 3)).reshape(
            _C_CHUNK, _B_CHUNK, hw)

        head = t[0:4]
        sig_head = jax.nn.sigmoid(head)
        ex_head = jnp.exp(head)
        special = jnp.where(
            r == 0, (sig_head + gx) * stride_w,
            jnp.where(
                r == 1, (sig_head + gy) * stride_h,
                jnp.where(
                    r == 2, ex_head * (aw8 * stride_w),
                    ex_head * (ah8 * stride_h))))
        head_val = jnp.where(cc == 0, special, sig_head)
        val = jnp.concatenate([head_val, jax.nn.sigmoid(t[4:])], axis=0)
        out_ref[:, :, pl.ds(a * hw, hw)] = val


def kernel(input):
    bs, ch, in_h, in_w = input.shape
    hw = in_h * in_w
    n = _NUM_ANCHORS * hw
    stride_h = _INPUT_SHAPE[0] / in_h
    stride_w = _INPUT_SHAPE[1] / in_w
    n_cc = _BBOX_ATTRS // _C_CHUNK
    n_bb = bs // _B_CHUNK

    def in_spec(a):
        return pl.BlockSpec(
            (_B_CHUNK, _C_CHUNK, in_h, in_w),
            lambda bb, cc, a=a: (bb, a * n_cc + cc, 0, 0))

    outT = pl.pallas_call(
        lambda i0, i1, i2, o_ref: _decode_kernel(
            i0, i1, i2, o_ref, in_h=in_h, in_w=in_w,
            stride_w=stride_w, stride_h=stride_h),
        grid=(n_bb, n_cc),
        in_specs=[in_spec(0), in_spec(1), in_spec(2)],
        out_specs=pl.BlockSpec(
            (_C_CHUNK, _B_CHUNK, n),
            lambda bb, cc: (cc, bb, 0)),
        out_shape=jax.ShapeDtypeStruct((_BBOX_ATTRS, bs, n), jnp.float32),
    )(input, input, input)
    return jnp.transpose(outT, (1, 2, 0))
